# Initial kernel scaffold; baseline (speedup 1.0000x reference)
import jax
import jax.numpy as jnp
from jax.experimental import pallas as pl


def kernel(phoneme_ids, padding_mask, table, pos_bias):
    raise NotImplementedError("write your pallas kernel here")


# speedup vs baseline: 1.5741x; 1.5741x over previous
import functools

import jax
import jax.numpy as jnp
from jax import lax
from jax.experimental import pallas as pl
from jax.experimental.pallas import tpu as pltpu
from jax.experimental.pallas import tpu_sc as plsc

NC, NS, L = 2, 16, 16
NW = NC * NS
IW = 128
G = 2
CHUNK = G * IW


def _build_sc_call(N, V, D, VROWS):
    n_per_w = N // NW
    n_chunks = n_per_w // CHUNK
    mesh = plsc.VectorSubcoreMesh(
        core_axis_name="c", subcore_axis_name="s",
        num_cores=NC, num_subcores=NS)

    @functools.partial(
        pl.kernel,
        out_type=jax.ShapeDtypeStruct((N, D), jnp.float32),
        mesh=mesh,
        scratch_types=[
            pltpu.HBM((VROWS, D), jnp.float32),
            pltpu.VMEM((VROWS, D), jnp.float32),
            pltpu.VMEM((D,), jnp.float32),
            pltpu.VMEM((G, IW), jnp.int32),
            pltpu.VMEM((G, IW), jnp.int32),
            pltpu.VMEM((G, IW), jnp.int32),
            pltpu.VMEM((CHUNK, D), jnp.float32),
            pltpu.SemaphoreType.DMA,
        ],
    )
    def sc_fn(ids_hbm, mask_hbm, table_hbm, bias_hbm, out_hbm,
              aug_hbm, aug_v, bias_v, idx_v, msk_v, eff_v, row_v, sem):
        cid = lax.axis_index("c")
        sid = lax.axis_index("s")
        wid = sid * NC + cid

        @pl.when(sid == 0)
        def _build():
            pltpu.sync_copy(table_hbm, aug_v.at[pl.ds(0, V)])
            pltpu.sync_copy(bias_hbm, bias_v)

            def add_bias(r, carry):
                for j in range(D // L):
                    sl = pl.ds(j * L, L)
                    aug_v[r, sl] = aug_v[r, sl] + bias_v[sl]
                return carry

            lax.fori_loop(0, V, add_bias, 0)
            zero = jnp.zeros((L,), jnp.float32)
            for r in range(V, VROWS):
                for j in range(D // L):
                    aug_v[r, pl.ds(j * L, L)] = zero
            pltpu.sync_copy(aug_v, aug_hbm)

        plsc.subcore_barrier()

        grows_per_w = n_per_w // IW

        def do_chunk(t, carry):
            grow = wid * grows_per_w + t * G
            pltpu.sync_copy(ids_hbm.at[pl.ds(grow, G)], idx_v)
            pltpu.sync_copy(mask_hbm.at[pl.ds(grow, G)], msk_v)

            def sel(k, c2):
                sl = pl.ds(k * L, L)
                for g in range(G):
                    idv = idx_v[g, sl]
                    mv = msk_v[g, sl]
                    eff_v[g, sl] = jnp.where(mv != 0, V, idv)
                return c2

            lax.fori_loop(0, IW // L, sel, 0)

            for g in range(G):
                pltpu.async_copy(
                    aug_hbm.at[eff_v.at[g]],
                    row_v.at[pl.ds(g * IW, IW)], sem).wait()
            pltpu.sync_copy(row_v, out_hbm.at[pl.ds(grow * IW, CHUNK)])
            return carry

        lax.fori_loop(0, n_chunks, do_chunk, 0)

    return sc_fn


def kernel(phoneme_ids, padding_mask, table, pos_bias):
    B, T = phoneme_ids.shape
    V, D = table.shape
    N = B * T
    VROWS = ((V + 1 + 7) // 8) * 8

    ids = phoneme_ids.reshape(N // IW, IW).astype(jnp.int32)
    mask = padding_mask.reshape(N // IW, IW).astype(jnp.int32)
    bias = pos_bias.reshape(D).astype(jnp.float32)

    sc_fn = _build_sc_call(N, V, D, VROWS)
    out = sc_fn(ids, mask, table, bias)
    return out.reshape(B, T, D)
